# Initial kernel scaffold; baseline (speedup 1.0000x reference)
import jax
import jax.numpy as jnp
from jax.experimental import pallas as pl


def kernel(state_one_hot, edges, edge_types, W_state, b_state, edge_emb, W0, b0, W1, b1):
    raise NotImplementedError("write your pallas kernel here")


# speedup vs baseline: 1.8122x; 1.8122x over previous
import functools

import jax
import jax.numpy as jnp
from jax import lax
from jax.experimental import pallas as pl
from jax.experimental.pallas import tpu as pltpu
from jax.experimental.pallas import tpu_sc as plsc

N = 10000
E = 320000
D = 128
S = 64
R = 32
EPS = 1e-6

NC = 2
NS = 16
NW = NC * NS
CH = 128
FLAT_E = 2 * E
IB = 16
G = 160
PE = NW * G * CH
EPW = G * CH
NPAD = 10240
RPS = NPAD // NS

BN = 1000


def _state_proj_body(soh_ref, wst_ref, bs_ref, wa_ref, b_ref, out_ref):
    x = soh_ref[...]
    cnt = jnp.sum(x, axis=1, keepdims=True) + EPS
    se = (jnp.dot(x, wst_ref[...], preferred_element_type=jnp.float32)
          + bs_ref[...]) / cnt
    out_ref[...] = jnp.dot(se, wa_ref[...],
                           preferred_element_type=jnp.float32) + b_ref[...]


def _state_proj(soh, wst_t, bs, wa_t, b):
    return pl.pallas_call(
        _state_proj_body,
        grid=(N // BN,),
        in_specs=[
            pl.BlockSpec((BN, S), lambda i: (i, 0)),
            pl.BlockSpec((S, D), lambda i: (0, 0)),
            pl.BlockSpec((1, D), lambda i: (0, 0)),
            pl.BlockSpec((D, D), lambda i: (0, 0)),
            pl.BlockSpec((1, D), lambda i: (0, 0)),
        ],
        out_specs=pl.BlockSpec((BN, D), lambda i: (i, 0)),
        out_shape=jax.ShapeDtypeStruct((N, D), jnp.float32),
    )(soh, wst_t, bs, wa_t, b)


def _etp_body(ee_ref, w0b_ref, w1b_ref, out_ref):
    e = ee_ref[...]
    out_ref[0:R, :] = jnp.dot(e, w0b_ref[...],
                              preferred_element_type=jnp.float32)
    out_ref[R:2 * R, :] = jnp.dot(e, w1b_ref[...],
                                  preferred_element_type=jnp.float32)


def _etp_tables(edge_emb, w0b_t, w1b_t):
    return pl.pallas_call(
        _etp_body,
        out_shape=jax.ShapeDtypeStruct((2 * R, D), jnp.float32),
    )(edge_emb, w0b_t, w1b_t)


def _sum_proj_body(acc_ref, wa_ref, b_ref, out_ref):
    h = acc_ref[0] + acc_ref[1]
    out_ref[...] = jnp.dot(h, wa_ref[...],
                           preferred_element_type=jnp.float32) + b_ref[...]


def _sum_proj(acc, wa_t, b):
    return pl.pallas_call(
        _sum_proj_body,
        grid=(N // BN,),
        in_specs=[
            pl.BlockSpec((2, BN, D), lambda i: (0, i, 0)),
            pl.BlockSpec((D, D), lambda i: (0, 0)),
            pl.BlockSpec((1, D), lambda i: (0, 0)),
        ],
        out_specs=pl.BlockSpec((BN, D), lambda i: (i, 0)),
        out_shape=jax.ShapeDtypeStruct((N, D), jnp.float32),
    )(acc, wa_t, b)


def _sum_body(acc_ref, out_ref):
    out_ref[...] = acc_ref[0] + acc_ref[1]


def _sum_parts(acc):
    return pl.pallas_call(
        _sum_body,
        grid=(N // BN,),
        in_specs=[pl.BlockSpec((2, BN, D), lambda i: (0, i, 0))],
        out_specs=pl.BlockSpec((BN, D), lambda i: (i, 0)),
        out_shape=jax.ShapeDtypeStruct((N, D), jnp.float32),
    )(acc)


def _edge_scatter_body(q_hbm, src_hbm, dst_hbm, out_hbm,
                       src_v, dst_v, rows_v, acc_sh, sem):
    c = lax.axis_index("c")
    s = lax.axis_index("s")
    w = s * NC + c

    zero = jnp.zeros((16,), jnp.float32)

    def zbody(i, carry):
        for j in range(D // 16):
            rows_v[i, pl.ds(j * 16, 16)] = zero
        return carry

    lax.fori_loop(0, CH, zbody, 0)
    for k in range(RPS // CH):
        pltpu.sync_copy(rows_v,
                        acc_sh.at[pl.ds(s * RPS + k * CH, CH)])
    plsc.subcore_barrier()

    def outer(ib, carry):
        pltpu.sync_copy(src_hbm.at[w, pl.ds(ib * IB, IB)], src_v)
        pltpu.sync_copy(dst_hbm.at[w, pl.ds(ib * IB, IB)], dst_v)

        def body(g, c2):
            pltpu.async_copy(q_hbm.at[src_v.at[g]], rows_v, sem).wait()
            pltpu.sync_copy(rows_v, acc_sh.at[dst_v.at[g]], add=True)
            return c2

        lax.fori_loop(0, IB, body, 0)
        return carry

    lax.fori_loop(0, G // IB, outer, 0)
    plsc.subcore_barrier()

    pltpu.sync_copy(acc_sh.at[pl.ds(s * RPS, RPS)],
                    out_hbm.at[c, pl.ds(s * RPS, RPS)])


_EDGE_SCATTER = functools.partial(
    pl.kernel,
    mesh=plsc.VectorSubcoreMesh(core_axis_name="c", subcore_axis_name="s"),
    out_type=jax.ShapeDtypeStruct((NC, NPAD, D), jnp.float32),
    scratch_types=[
        pltpu.VMEM((IB, CH), jnp.int32),
        pltpu.VMEM((IB, CH), jnp.int32),
        pltpu.VMEM((CH, D), jnp.float32),
        pltpu.VMEM_SHARED((NPAD, D), jnp.float32),
        pltpu.SemaphoreType.DMA,
    ],
)(_edge_scatter_body)


@jax.jit
def kernel(state_one_hot, edges, edge_types, W_state, b_state, edge_emb,
           W0, b0, W1, b1):
    wst_t = W_state.T
    w0a_t = W0[:, :D].T
    w0b_t = W0[:, D:].T
    w1a_t = W1[:, :D].T
    w1b_t = W1[:, D:].T
    bs2 = b_state.reshape(1, D)
    b0_2 = b0.reshape(1, D)
    b1_2 = b1.reshape(1, D)

    src = edges[0, :, 0].astype(jnp.int32)
    dst = edges[0, :, 1].astype(jnp.int32)
    et = edge_types[0].astype(jnp.int32)

    pad = PE - FLAT_E
    src2 = jnp.concatenate(
        [src, N + et, jnp.full((pad,), N + R, jnp.int32)]).reshape(NW, G, CH)
    dst2 = jnp.concatenate(
        [dst, dst, jnp.zeros((pad,), jnp.int32)]).reshape(NW, G, CH)

    etp = _etp_tables(edge_emb, w0b_t, w1b_t)
    zrow = jnp.zeros((1, D), jnp.float32)

    p0 = _state_proj(state_one_hot, wst_t, bs2, w0a_t, b0_2)
    q0 = jnp.concatenate([p0, etp[:R], zrow], axis=0)
    acc1 = _EDGE_SCATTER(q0, src2, dst2)

    p1 = _sum_proj(acc1, w1a_t, b1_2)
    q1 = jnp.concatenate([p1, etp[R:], zrow], axis=0)
    acc2 = _EDGE_SCATTER(q1, src2, dst2)

    node_embeddings = _sum_parts(acc2)
    return (node_embeddings, node_embeddings[0])


# speedup vs baseline: 1.9274x; 1.0636x over previous
import functools

import jax
import jax.numpy as jnp
from jax import lax
from jax.experimental import pallas as pl
from jax.experimental.pallas import tpu as pltpu
from jax.experimental.pallas import tpu_sc as plsc

N = 10000
E = 320000
D = 128
S = 64
R = 32
EPS = 1e-6

NC = 2
NS = 16
NW = NC * NS
CH = 128
FLAT_E = 2 * E
IB = 16
G = 160
PE = NW * G * CH
EPW = G * CH
NPAD = 10240
RPS = NPAD // NS

BN = 1000


def _state_proj_body(soh_ref, wst_ref, bs_ref, wa_ref, b_ref, out_ref):
    x = soh_ref[...]
    cnt = jnp.sum(x, axis=1, keepdims=True) + EPS
    se = (jnp.dot(x, wst_ref[...], preferred_element_type=jnp.float32)
          + bs_ref[...]) / cnt
    out_ref[...] = jnp.dot(se, wa_ref[...],
                           preferred_element_type=jnp.float32) + b_ref[...]


def _state_proj(soh, wst_t, bs, wa_t, b):
    return pl.pallas_call(
        _state_proj_body,
        grid=(N // BN,),
        in_specs=[
            pl.BlockSpec((BN, S), lambda i: (i, 0)),
            pl.BlockSpec((S, D), lambda i: (0, 0)),
            pl.BlockSpec((1, D), lambda i: (0, 0)),
            pl.BlockSpec((D, D), lambda i: (0, 0)),
            pl.BlockSpec((1, D), lambda i: (0, 0)),
        ],
        out_specs=pl.BlockSpec((BN, D), lambda i: (i, 0)),
        out_shape=jax.ShapeDtypeStruct((N, D), jnp.float32),
    )(soh, wst_t, bs, wa_t, b)


def _etp_body(ee_ref, w0b_ref, w1b_ref, out_ref):
    e = ee_ref[...]
    out_ref[0:R, :] = jnp.dot(e, w0b_ref[...],
                              preferred_element_type=jnp.float32)
    out_ref[R:2 * R, :] = jnp.dot(e, w1b_ref[...],
                                  preferred_element_type=jnp.float32)


def _etp_tables(edge_emb, w0b_t, w1b_t):
    return pl.pallas_call(
        _etp_body,
        out_shape=jax.ShapeDtypeStruct((2 * R, D), jnp.float32),
    )(edge_emb, w0b_t, w1b_t)


def _sum_proj_body(acc_ref, wa_ref, b_ref, out_ref):
    h = acc_ref[0] + acc_ref[1]
    out_ref[...] = jnp.dot(h, wa_ref[...],
                           preferred_element_type=jnp.float32) + b_ref[...]


def _sum_proj(acc, wa_t, b):
    return pl.pallas_call(
        _sum_proj_body,
        grid=(N // BN,),
        in_specs=[
            pl.BlockSpec((2, BN, D), lambda i: (0, i, 0)),
            pl.BlockSpec((D, D), lambda i: (0, 0)),
            pl.BlockSpec((1, D), lambda i: (0, 0)),
        ],
        out_specs=pl.BlockSpec((BN, D), lambda i: (i, 0)),
        out_shape=jax.ShapeDtypeStruct((N, D), jnp.float32),
    )(acc, wa_t, b)


def _sum_body(acc_ref, out_ref):
    out_ref[...] = acc_ref[0] + acc_ref[1]


def _sum_parts(acc):
    return pl.pallas_call(
        _sum_body,
        grid=(N // BN,),
        in_specs=[pl.BlockSpec((2, BN, D), lambda i: (0, i, 0))],
        out_specs=pl.BlockSpec((BN, D), lambda i: (i, 0)),
        out_shape=jax.ShapeDtypeStruct((N, D), jnp.float32),
    )(acc)


def _edge_scatter_body(q_hbm, src_hbm, dst_hbm, out_hbm,
                       src_v, dst_v, rows_v, acc_sh, sem0, sem1):
    c = lax.axis_index("c")
    s = lax.axis_index("s")
    w = s * NC + c

    zero = jnp.zeros((16,), jnp.float32)

    def zbody(i, carry):
        for j in range(D // 16):
            rows_v[0, i, pl.ds(j * 16, 16)] = zero
        return carry

    lax.fori_loop(0, CH, zbody, 0)
    for k in range(RPS // CH):
        pltpu.sync_copy(rows_v.at[0],
                        acc_sh.at[pl.ds(s * RPS + k * CH, CH)])
    plsc.subcore_barrier()

    sems = (sem0, sem1)

    def outer(ib, carry):
        pltpu.sync_copy(src_hbm.at[w, pl.ds(ib * IB, IB)], src_v)
        pltpu.sync_copy(dst_hbm.at[w, pl.ds(ib * IB, IB)], dst_v)

        cps = [None, None]
        cps[0] = pltpu.async_copy(q_hbm.at[src_v.at[0]], rows_v.at[0],
                                  sems[0])
        for j in range(IB):
            cur = j % 2
            cps[cur].wait()
            if j + 1 < IB:
                nxt = (j + 1) % 2
                cps[nxt] = pltpu.async_copy(
                    q_hbm.at[src_v.at[j + 1]], rows_v.at[nxt], sems[nxt])
            pltpu.sync_copy(rows_v.at[cur], acc_sh.at[dst_v.at[j]],
                            add=True)
        return carry

    lax.fori_loop(0, G // IB, outer, 0)
    plsc.subcore_barrier()

    pltpu.sync_copy(acc_sh.at[pl.ds(s * RPS, RPS)],
                    out_hbm.at[c, pl.ds(s * RPS, RPS)])


_EDGE_SCATTER = functools.partial(
    pl.kernel,
    mesh=plsc.VectorSubcoreMesh(core_axis_name="c", subcore_axis_name="s"),
    out_type=jax.ShapeDtypeStruct((NC, NPAD, D), jnp.float32),
    scratch_types=[
        pltpu.VMEM((IB, CH), jnp.int32),
        pltpu.VMEM((IB, CH), jnp.int32),
        pltpu.VMEM((2, CH, D), jnp.float32),
        pltpu.VMEM_SHARED((NPAD, D), jnp.float32),
        pltpu.SemaphoreType.DMA,
        pltpu.SemaphoreType.DMA,
    ],
)(_edge_scatter_body)


@jax.jit
def kernel(state_one_hot, edges, edge_types, W_state, b_state, edge_emb,
           W0, b0, W1, b1):
    wst_t = W_state.T
    w0a_t = W0[:, :D].T
    w0b_t = W0[:, D:].T
    w1a_t = W1[:, :D].T
    w1b_t = W1[:, D:].T
    bs2 = b_state.reshape(1, D)
    b0_2 = b0.reshape(1, D)
    b1_2 = b1.reshape(1, D)

    src = edges[0, :, 0].astype(jnp.int32)
    dst = edges[0, :, 1].astype(jnp.int32)
    et = edge_types[0].astype(jnp.int32)

    pad = PE - FLAT_E
    src2 = jnp.concatenate(
        [src, N + et, jnp.full((pad,), N + R, jnp.int32)]).reshape(NW, G, CH)
    dst2 = jnp.concatenate(
        [dst, dst, jnp.zeros((pad,), jnp.int32)]).reshape(NW, G, CH)

    etp = _etp_tables(edge_emb, w0b_t, w1b_t)
    zrow = jnp.zeros((1, D), jnp.float32)

    p0 = _state_proj(state_one_hot, wst_t, bs2, w0a_t, b0_2)
    q0 = jnp.concatenate([p0, etp[:R], zrow], axis=0)
    acc1 = _EDGE_SCATTER(q0, src2, dst2)

    p1 = _sum_proj(acc1, w1a_t, b1_2)
    q1 = jnp.concatenate([p1, etp[R:], zrow], axis=0)
    acc2 = _EDGE_SCATTER(q1, src2, dst2)

    node_embeddings = _sum_parts(acc2)
    return (node_embeddings, node_embeddings[0])


# speedup vs baseline: 2.2386x; 1.1614x over previous
import functools

import jax
import jax.numpy as jnp
from jax import lax
from jax.experimental import pallas as pl
from jax.experimental.pallas import tpu as pltpu
from jax.experimental.pallas import tpu_sc as plsc

N = 10000
E = 320000
D = 128
S = 64
R = 32
EPS = 1e-6

NC = 2
NS = 16
NW = NC * NS
CH = 64
FLAT_E = 2 * E
IB = 16
G = 320
NBUF = 4
PE = NW * G * CH
EPW = G * CH
NPAD = 10240
RPS = NPAD // NS

BN = 1000


def _state_proj_body(soh_ref, wst_ref, bs_ref, wa_ref, b_ref, out_ref):
    x = soh_ref[...]
    cnt = jnp.sum(x, axis=1, keepdims=True) + EPS
    se = (jnp.dot(x, wst_ref[...], preferred_element_type=jnp.float32)
          + bs_ref[...]) / cnt
    out_ref[...] = jnp.dot(se, wa_ref[...],
                           preferred_element_type=jnp.float32) + b_ref[...]


def _state_proj(soh, wst_t, bs, wa_t, b):
    return pl.pallas_call(
        _state_proj_body,
        grid=(N // BN,),
        in_specs=[
            pl.BlockSpec((BN, S), lambda i: (i, 0)),
            pl.BlockSpec((S, D), lambda i: (0, 0)),
            pl.BlockSpec((1, D), lambda i: (0, 0)),
            pl.BlockSpec((D, D), lambda i: (0, 0)),
            pl.BlockSpec((1, D), lambda i: (0, 0)),
        ],
        out_specs=pl.BlockSpec((BN, D), lambda i: (i, 0)),
        out_shape=jax.ShapeDtypeStruct((N, D), jnp.float32),
    )(soh, wst_t, bs, wa_t, b)


def _etp_body(ee_ref, w0b_ref, w1b_ref, out_ref):
    e = ee_ref[...]
    out_ref[0:R, :] = jnp.dot(e, w0b_ref[...],
                              preferred_element_type=jnp.float32)
    out_ref[R:2 * R, :] = jnp.dot(e, w1b_ref[...],
                                  preferred_element_type=jnp.float32)


def _etp_tables(edge_emb, w0b_t, w1b_t):
    return pl.pallas_call(
        _etp_body,
        out_shape=jax.ShapeDtypeStruct((2 * R, D), jnp.float32),
    )(edge_emb, w0b_t, w1b_t)


def _sum_proj_body(acc_ref, wa_ref, b_ref, out_ref):
    h = acc_ref[0] + acc_ref[1]
    out_ref[...] = jnp.dot(h, wa_ref[...],
                           preferred_element_type=jnp.float32) + b_ref[...]


def _sum_proj(acc, wa_t, b):
    return pl.pallas_call(
        _sum_proj_body,
        grid=(N // BN,),
        in_specs=[
            pl.BlockSpec((2, BN, D), lambda i: (0, i, 0)),
            pl.BlockSpec((D, D), lambda i: (0, 0)),
            pl.BlockSpec((1, D), lambda i: (0, 0)),
        ],
        out_specs=pl.BlockSpec((BN, D), lambda i: (i, 0)),
        out_shape=jax.ShapeDtypeStruct((N, D), jnp.float32),
    )(acc, wa_t, b)


def _sum_body(acc_ref, out_ref):
    out_ref[...] = acc_ref[0] + acc_ref[1]


def _sum_parts(acc):
    return pl.pallas_call(
        _sum_body,
        grid=(N // BN,),
        in_specs=[pl.BlockSpec((2, BN, D), lambda i: (0, i, 0))],
        out_specs=pl.BlockSpec((BN, D), lambda i: (i, 0)),
        out_shape=jax.ShapeDtypeStruct((N, D), jnp.float32),
    )(acc)


def _edge_scatter_body(q_hbm, src_hbm, dst_hbm, out_hbm,
                       src_v, dst_v, rows_v, acc_sh, *sems):
    c = lax.axis_index("c")
    s = lax.axis_index("s")
    w = s * NC + c
    gsem = sems[:NBUF]
    ssem = sems[NBUF:]

    zero = jnp.zeros((16,), jnp.float32)

    def zbody(i, carry):
        for j in range(D // 16):
            rows_v[0, i, pl.ds(j * 16, 16)] = zero
        return carry

    lax.fori_loop(0, CH, zbody, 0)
    for k in range(RPS // CH):
        pltpu.sync_copy(rows_v.at[0],
                        acc_sh.at[pl.ds(s * RPS + k * CH, CH)])
    plsc.subcore_barrier()

    def outer(ib, carry):
        pltpu.sync_copy(src_hbm.at[w, pl.ds(ib * IB, IB)], src_v)
        pltpu.sync_copy(dst_hbm.at[w, pl.ds(ib * IB, IB)], dst_v)

        gcp = [None] * IB
        scp = [None] * IB
        for k in range(2):
            gcp[k] = pltpu.async_copy(q_hbm.at[src_v.at[k]],
                                      rows_v.at[k], gsem[k])
        for j in range(IB):
            b = j % NBUF
            gcp[j].wait()
            scp[j] = pltpu.async_copy(rows_v.at[b],
                                      acc_sh.at[dst_v.at[j]], ssem[b],
                                      add=True)
            if j + 2 < IB:
                if j - 2 >= 0:
                    scp[j - 2].wait()
                b2 = (j + 2) % NBUF
                gcp[j + 2] = pltpu.async_copy(q_hbm.at[src_v.at[j + 2]],
                                              rows_v.at[b2], gsem[b2])
        for k in range(max(0, IB - 4), IB):
            if scp[k] is not None and k > IB - 5:
                scp[k].wait()
        return carry

    lax.fori_loop(0, G // IB, outer, 0)
    plsc.subcore_barrier()

    pltpu.sync_copy(acc_sh.at[pl.ds(s * RPS, RPS)],
                    out_hbm.at[c, pl.ds(s * RPS, RPS)])


_EDGE_SCATTER = functools.partial(
    pl.kernel,
    mesh=plsc.VectorSubcoreMesh(core_axis_name="c", subcore_axis_name="s"),
    out_type=jax.ShapeDtypeStruct((NC, NPAD, D), jnp.float32),
    scratch_types=[
        pltpu.VMEM((IB, CH), jnp.int32),
        pltpu.VMEM((IB, CH), jnp.int32),
        pltpu.VMEM((NBUF, CH, D), jnp.float32),
        pltpu.VMEM_SHARED((NPAD, D), jnp.float32),
    ] + [pltpu.SemaphoreType.DMA] * (2 * NBUF),
)(_edge_scatter_body)


@jax.jit
def kernel(state_one_hot, edges, edge_types, W_state, b_state, edge_emb,
           W0, b0, W1, b1):
    wst_t = W_state.T
    w0a_t = W0[:, :D].T
    w0b_t = W0[:, D:].T
    w1a_t = W1[:, :D].T
    w1b_t = W1[:, D:].T
    bs2 = b_state.reshape(1, D)
    b0_2 = b0.reshape(1, D)
    b1_2 = b1.reshape(1, D)

    src = edges[0, :, 0].astype(jnp.int32)
    dst = edges[0, :, 1].astype(jnp.int32)
    et = edge_types[0].astype(jnp.int32)

    pad = PE - FLAT_E
    src2 = jnp.concatenate(
        [src, N + et, jnp.full((pad,), N + R, jnp.int32)]).reshape(NW, G, CH)
    dst2 = jnp.concatenate(
        [dst, dst, jnp.zeros((pad,), jnp.int32)]).reshape(NW, G, CH)

    etp = _etp_tables(edge_emb, w0b_t, w1b_t)
    zrow = jnp.zeros((1, D), jnp.float32)

    p0 = _state_proj(state_one_hot, wst_t, bs2, w0a_t, b0_2)
    q0 = jnp.concatenate([p0, etp[:R], zrow], axis=0)
    acc1 = _EDGE_SCATTER(q0, src2, dst2)

    p1 = _sum_proj(acc1, w1a_t, b1_2)
    q1 = jnp.concatenate([p1, etp[R:], zrow], axis=0)
    acc2 = _EDGE_SCATTER(q1, src2, dst2)

    node_embeddings = _sum_parts(acc2)
    return (node_embeddings, node_embeddings[0])


# speedup vs baseline: 2.6939x; 1.2034x over previous
import functools

import jax
import jax.numpy as jnp
from jax import lax
from jax.experimental import pallas as pl
from jax.experimental.pallas import tpu as pltpu
from jax.experimental.pallas import tpu_sc as plsc

N = 10000
E = 320000
D = 128
S = 64
R = 32
EPS = 1e-6

NC = 2
NS = 16
NW = NC * NS
CH = 64
IB = 32
G = 160
NBUF = 5
GF = 3
PE = NW * G * CH
NPAD = 10240
RPS = NPAD // NS

BN = 1000


def _state_proj_body(soh_ref, wst_ref, bs_ref, wa_ref, out_ref):
    x = soh_ref[...]
    cnt = jnp.sum(x, axis=1, keepdims=True) + EPS
    se = (jnp.dot(x, wst_ref[...], preferred_element_type=jnp.float32)
          + bs_ref[...]) / cnt
    out_ref[...] = jnp.dot(se, wa_ref[...],
                           preferred_element_type=jnp.float32)


def _state_proj(soh, wst_t, bs, wa_t):
    return pl.pallas_call(
        _state_proj_body,
        grid=(N // BN,),
        in_specs=[
            pl.BlockSpec((BN, S), lambda i: (i, 0)),
            pl.BlockSpec((S, D), lambda i: (0, 0)),
            pl.BlockSpec((1, D), lambda i: (0, 0)),
            pl.BlockSpec((D, D), lambda i: (0, 0)),
        ],
        out_specs=pl.BlockSpec((BN, D), lambda i: (i, 0)),
        out_shape=jax.ShapeDtypeStruct((N, D), jnp.float32),
    )(soh, wst_t, bs, wa_t)


def _etp_body(ee_ref, w0b_ref, b0_ref, w1b_ref, b1_ref, out_ref):
    e = ee_ref[...]
    out_ref[0:R, :] = jnp.dot(e, w0b_ref[...],
                              preferred_element_type=jnp.float32) + b0_ref[...]
    out_ref[R:2 * R, :] = jnp.dot(e, w1b_ref[...],
                                  preferred_element_type=jnp.float32) + b1_ref[...]


def _etp_tables(edge_emb, w0b_t, b0, w1b_t, b1):
    return pl.pallas_call(
        _etp_body,
        out_shape=jax.ShapeDtypeStruct((2 * R, D), jnp.float32),
    )(edge_emb, w0b_t, b0, w1b_t, b1)


def _mid_body(acc_ref, deg_ref, etp_ref, wa_ref, out_ref):
    h = (acc_ref[0] + acc_ref[1]
         + jnp.dot(deg_ref[0] + deg_ref[1], etp_ref[...],
                   preferred_element_type=jnp.float32))
    out_ref[...] = jnp.dot(h, wa_ref[...],
                           preferred_element_type=jnp.float32)


def _mid_proj(acc, deg, etp, wa_t):
    return pl.pallas_call(
        _mid_body,
        grid=(N // BN,),
        in_specs=[
            pl.BlockSpec((2, BN, D), lambda i: (0, i, 0)),
            pl.BlockSpec((2, BN, D), lambda i: (0, i, 0)),
            pl.BlockSpec((D, D), lambda i: (0, 0)),
            pl.BlockSpec((D, D), lambda i: (0, 0)),
        ],
        out_specs=pl.BlockSpec((BN, D), lambda i: (i, 0)),
        out_shape=jax.ShapeDtypeStruct((N, D), jnp.float32),
    )(acc, deg, etp, wa_t)


def _fin_body(acc_ref, deg_ref, etp_ref, out_ref):
    out_ref[...] = (acc_ref[0] + acc_ref[1]
                    + jnp.dot(deg_ref[0] + deg_ref[1], etp_ref[...],
                              preferred_element_type=jnp.float32))


def _fin_sum(acc, deg, etp):
    return pl.pallas_call(
        _fin_body,
        grid=(N // BN,),
        in_specs=[
            pl.BlockSpec((2, BN, D), lambda i: (0, i, 0)),
            pl.BlockSpec((2, BN, D), lambda i: (0, i, 0)),
            pl.BlockSpec((D, D), lambda i: (0, 0)),
        ],
        out_specs=pl.BlockSpec((BN, D), lambda i: (i, 0)),
        out_shape=jax.ShapeDtypeStruct((N, D), jnp.float32),
    )(acc, deg, etp)


def _make_scatter(width):
    def body(q_hbm, src_hbm, dst_hbm, out_hbm,
             src_v, dst_v, rows_v, acc_sh, *sems):
        c = lax.axis_index("c")
        s = lax.axis_index("s")
        w = s * NC + c
        gsem = sems[:NBUF]
        ssem = sems[NBUF:]
        q_src = q_hbm

        zero = jnp.zeros((16,), jnp.float32)

        def zbody(i, carry):
            for j in range(width // 16):
                rows_v[0, i, pl.ds(j * 16, 16)] = zero
            return carry

        lax.fori_loop(0, CH, zbody, 0)
        zcp = []
        for k in range(RPS // CH):
            zcp.append(pltpu.async_copy(
                rows_v.at[0], acc_sh.at[pl.ds(s * RPS + k * CH, CH)],
                ssem[k % (NBUF - GF)]))
            if k >= NBUF - GF:
                zcp[k - (NBUF - GF)].wait()
        for k in range(RPS // CH - (NBUF - GF), RPS // CH):
            zcp[k].wait()
        plsc.subcore_barrier()

        def outer(ib, carry):
            pltpu.sync_copy(src_hbm.at[w, pl.ds(ib * IB, IB)], src_v)
            pltpu.sync_copy(dst_hbm.at[w, pl.ds(ib * IB, IB)], dst_v)

            gcp = [None] * IB
            scp = [None] * IB
            for k in range(GF):
                gcp[k] = pltpu.async_copy(q_src.at[src_v.at[k]],
                                          rows_v.at[k], gsem[k])
            for j in range(IB):
                b = j % NBUF
                gcp[j].wait()
                scp[j] = pltpu.async_copy(rows_v.at[b],
                                          acc_sh.at[dst_v.at[j]], ssem[b],
                                          add=True)
                if j + GF < IB:
                    if j + GF - NBUF >= 0:
                        scp[j + GF - NBUF].wait()
                    b2 = (j + GF) % NBUF
                    gcp[j + GF] = pltpu.async_copy(q_src.at[src_v.at[j + GF]],
                                                   rows_v.at[b2], gsem[b2])
            for k in range(max(0, IB - NBUF), IB):
                if scp[k] is not None:
                    scp[k].wait()
            return carry

        lax.fori_loop(0, G // IB, outer, 0)
        plsc.subcore_barrier()

        pltpu.sync_copy(acc_sh.at[pl.ds(s * RPS, RPS)],
                        out_hbm.at[c, pl.ds(s * RPS, RPS)])

    scratch = [
        pltpu.VMEM((IB, CH), jnp.int32),
        pltpu.VMEM((IB, CH), jnp.int32),
        pltpu.VMEM((NBUF, CH, width), jnp.float32),
        pltpu.VMEM_SHARED((NPAD, width), jnp.float32),
    ]
    return pl.kernel(
        body,
        mesh=plsc.VectorSubcoreMesh(core_axis_name="c", subcore_axis_name="s"),
        out_type=jax.ShapeDtypeStruct((NC, NPAD, width), jnp.float32),
        scratch_types=scratch + [pltpu.SemaphoreType.DMA] * (2 * NBUF),
    )


_EDGE_SCATTER = _make_scatter(D)


@jax.jit
def kernel(state_one_hot, edges, edge_types, W_state, b_state, edge_emb,
           W0, b0, W1, b1):
    wst_t = W_state.T
    w0a_t = W0[:, :D].T
    w0b_t = W0[:, D:].T
    w1a_t = W1[:, :D].T
    w1b_t = W1[:, D:].T
    bs2 = b_state.reshape(1, D)
    b0_2 = b0.reshape(1, D)
    b1_2 = b1.reshape(1, D)

    src = edges[0, :, 0].astype(jnp.int32)
    dst = edges[0, :, 1].astype(jnp.int32)
    et = edge_types[0].astype(jnp.int32)

    pad = PE - E
    src2 = jnp.concatenate(
        [src, jnp.full((pad,), N, jnp.int32)]).reshape(NW, G, CH)
    et2 = jnp.concatenate(
        [et, jnp.full((pad,), R, jnp.int32)]).reshape(NW, G, CH)
    dst2 = jnp.concatenate(
        [dst, jnp.zeros((pad,), jnp.int32)]).reshape(NW, G, CH)

    ident = jnp.concatenate(
        [jnp.eye(R, D, dtype=jnp.float32), jnp.zeros((1, D), jnp.float32)])
    deg = _EDGE_SCATTER(ident, et2, dst2)

    etp = _etp_tables(edge_emb, w0b_t, b0_2, w1b_t, b1_2)
    zpad = jnp.zeros((D - R, D), jnp.float32)
    etp0 = jnp.concatenate([etp[:R], zpad], axis=0)
    etp1 = jnp.concatenate([etp[R:], zpad], axis=0)
    zrow = jnp.zeros((1, D), jnp.float32)

    p0 = _state_proj(state_one_hot, wst_t, bs2, w0a_t)
    q0 = jnp.concatenate([p0, zrow], axis=0)
    acc1 = _EDGE_SCATTER(q0, src2, dst2)

    p1 = _mid_proj(acc1, deg, etp0, w1a_t)
    q1 = jnp.concatenate([p1, zrow], axis=0)
    acc2 = _EDGE_SCATTER(q1, src2, dst2)

    node_embeddings = _fin_sum(acc2, deg, etp1)
    return (node_embeddings, node_embeddings[0])


# speedup vs baseline: 2.7580x; 1.0238x over previous
import functools

import jax
import jax.numpy as jnp
from jax import lax
from jax.experimental import pallas as pl
from jax.experimental.pallas import tpu as pltpu
from jax.experimental.pallas import tpu_sc as plsc

N = 10000
E = 320000
D = 128
S = 64
R = 32
EPS = 1e-6

NC = 2
NS = 16
NW = NC * NS
CH = 64
IB = 32
G = 160
NBUF = 5
GF = 3
PE = NW * G * CH
NPAD = 10240
RPS = NPAD // NS

BN = 1000


def _state_proj_body(soh_ref, wst_ref, bs_ref, wa_ref, out_ref):
    x = soh_ref[...]
    cnt = jnp.sum(x, axis=1, keepdims=True) + EPS
    se = (jnp.dot(x, wst_ref[...], preferred_element_type=jnp.float32)
          + bs_ref[...]) / cnt
    out_ref[...] = jnp.dot(se, wa_ref[...],
                           preferred_element_type=jnp.float32)


def _state_proj(soh, wst_t, bs, wa_t):
    return pl.pallas_call(
        _state_proj_body,
        grid=(N // BN,),
        in_specs=[
            pl.BlockSpec((BN, S), lambda i: (i, 0)),
            pl.BlockSpec((S, D), lambda i: (0, 0)),
            pl.BlockSpec((1, D), lambda i: (0, 0)),
            pl.BlockSpec((D, D), lambda i: (0, 0)),
        ],
        out_specs=pl.BlockSpec((BN, D), lambda i: (i, 0)),
        out_shape=jax.ShapeDtypeStruct((N, D), jnp.float32),
    )(soh, wst_t, bs, wa_t)


def _etp_body(ee_ref, w0b_ref, b0_ref, w1b_ref, b1_ref, out_ref):
    e = ee_ref[...]
    out_ref[0:R, :] = jnp.dot(e, w0b_ref[...],
                              preferred_element_type=jnp.float32) + b0_ref[...]
    out_ref[R:2 * R, :] = jnp.dot(e, w1b_ref[...],
                                  preferred_element_type=jnp.float32) + b1_ref[...]


def _etp_tables(edge_emb, w0b_t, b0, w1b_t, b1):
    return pl.pallas_call(
        _etp_body,
        out_shape=jax.ShapeDtypeStruct((2 * R, D), jnp.float32),
    )(edge_emb, w0b_t, b0, w1b_t, b1)


def _mid_body(acc_ref, deg_ref, etp_ref, wa_ref, out_ref):
    h = (acc_ref[0] + acc_ref[1]
         + jnp.dot(deg_ref[0] + deg_ref[1], etp_ref[...],
                   preferred_element_type=jnp.float32))
    out_ref[...] = jnp.dot(h, wa_ref[...],
                           preferred_element_type=jnp.float32)


def _mid_proj(acc, deg, etp, wa_t):
    return pl.pallas_call(
        _mid_body,
        grid=(N // BN,),
        in_specs=[
            pl.BlockSpec((2, BN, D), lambda i: (0, i, 0)),
            pl.BlockSpec((2, BN, D), lambda i: (0, i, 0)),
            pl.BlockSpec((D, D), lambda i: (0, 0)),
            pl.BlockSpec((D, D), lambda i: (0, 0)),
        ],
        out_specs=pl.BlockSpec((BN, D), lambda i: (i, 0)),
        out_shape=jax.ShapeDtypeStruct((N, D), jnp.float32),
    )(acc, deg, etp, wa_t)


def _fin_body(acc_ref, deg_ref, etp_ref, out_ref):
    out_ref[...] = (acc_ref[0] + acc_ref[1]
                    + jnp.dot(deg_ref[0] + deg_ref[1], etp_ref[...],
                              preferred_element_type=jnp.float32))


def _fin_sum(acc, deg, etp):
    return pl.pallas_call(
        _fin_body,
        grid=(N // BN,),
        in_specs=[
            pl.BlockSpec((2, BN, D), lambda i: (0, i, 0)),
            pl.BlockSpec((2, BN, D), lambda i: (0, i, 0)),
            pl.BlockSpec((D, D), lambda i: (0, 0)),
        ],
        out_specs=pl.BlockSpec((BN, D), lambda i: (i, 0)),
        out_shape=jax.ShapeDtypeStruct((N, D), jnp.float32),
    )(acc, deg, etp)


def _zero_acc(s, rows_v, acc_sh, ssem):
    zero = jnp.zeros((16,), jnp.float32)

    def zbody(i, carry):
        for j in range(D // 16):
            rows_v[0, i, pl.ds(j * 16, 16)] = zero
        return carry

    lax.fori_loop(0, CH, zbody, 0)
    zcp = []
    for k in range(RPS // CH):
        zcp.append(pltpu.async_copy(
            rows_v.at[0], acc_sh.at[pl.ds(s * RPS + k * CH, CH)],
            ssem[k % (NBUF - GF)]))
        if k >= NBUF - GF:
            zcp[k - (NBUF - GF)].wait()
    for k in range(RPS // CH - (NBUF - GF), RPS // CH):
        zcp[k].wait()


def _scatter_pass(w, q_src, src_hbm, dst_hbm,
                  src_v, dst_v, rows_v, acc_sh, gsem, ssem):
    def outer(ib, carry):
        pltpu.sync_copy(src_hbm.at[w, pl.ds(ib * IB, IB)], src_v)
        pltpu.sync_copy(dst_hbm.at[w, pl.ds(ib * IB, IB)], dst_v)

        gcp = [None] * IB
        scp = [None] * IB
        for k in range(GF):
            gcp[k] = pltpu.async_copy(q_src.at[src_v.at[k]],
                                      rows_v.at[k], gsem[k])
        for j in range(IB):
            b = j % NBUF
            gcp[j].wait()
            scp[j] = pltpu.async_copy(rows_v.at[b],
                                      acc_sh.at[dst_v.at[j]], ssem[b],
                                      add=True)
            if j + GF < IB:
                if j + GF - NBUF >= 0:
                    scp[j + GF - NBUF].wait()
                b2 = (j + GF) % NBUF
                gcp[j + GF] = pltpu.async_copy(q_src.at[src_v.at[j + GF]],
                                               rows_v.at[b2], gsem[b2])
        for k in range(max(0, IB - NBUF), IB):
            if scp[k] is not None:
                scp[k].wait()
        return carry

    lax.fori_loop(0, G // IB, outer, 0)


def _export_acc(c, s, acc_sh, out_hbm):
    pltpu.sync_copy(acc_sh.at[pl.ds(s * RPS, RPS)],
                    out_hbm.at[c, pl.ds(s * RPS, RPS)])


_SCRATCH = [
    pltpu.VMEM((IB, CH), jnp.int32),
    pltpu.VMEM((IB, CH), jnp.int32),
    pltpu.VMEM((NBUF, CH, D), jnp.float32),
    pltpu.VMEM_SHARED((NPAD, D), jnp.float32),
]
_MESH = plsc.VectorSubcoreMesh(core_axis_name="c", subcore_axis_name="s")
_OUT = jax.ShapeDtypeStruct((NC, NPAD, D), jnp.float32)


def _edge_scatter_body(q_hbm, src_hbm, dst_hbm, out_hbm,
                       src_v, dst_v, rows_v, acc_sh, *sems):
    c = lax.axis_index("c")
    s = lax.axis_index("s")
    w = s * NC + c
    gsem = sems[:NBUF]
    ssem = sems[NBUF:]
    _zero_acc(s, rows_v, acc_sh, ssem)
    plsc.subcore_barrier()
    _scatter_pass(w, q_hbm, src_hbm, dst_hbm,
                  src_v, dst_v, rows_v, acc_sh, gsem, ssem)
    plsc.subcore_barrier()
    _export_acc(c, s, acc_sh, out_hbm)


_EDGE_SCATTER = pl.kernel(
    _edge_scatter_body,
    mesh=_MESH,
    out_type=_OUT,
    scratch_types=_SCRATCH + [pltpu.SemaphoreType.DMA] * (2 * NBUF),
)


def _fused_body(tbl_hbm, q_hbm, et_hbm, src_hbm, dst_hbm,
                deg_hbm, out_hbm,
                src_v, dst_v, rows_v, acc_sh, *sems):
    c = lax.axis_index("c")
    s = lax.axis_index("s")
    w = s * NC + c
    gsem = sems[:NBUF]
    ssem = sems[NBUF:]
    _zero_acc(s, rows_v, acc_sh, ssem)
    plsc.subcore_barrier()
    _scatter_pass(w, tbl_hbm, et_hbm, dst_hbm,
                  src_v, dst_v, rows_v, acc_sh, gsem, ssem)
    plsc.subcore_barrier()
    _export_acc(c, s, acc_sh, deg_hbm)
    _zero_acc(s, rows_v, acc_sh, ssem)
    plsc.subcore_barrier()
    _scatter_pass(w, q_hbm, src_hbm, dst_hbm,
                  src_v, dst_v, rows_v, acc_sh, gsem, ssem)
    plsc.subcore_barrier()
    _export_acc(c, s, acc_sh, out_hbm)


_FUSED_SCATTER = pl.kernel(
    _fused_body,
    mesh=_MESH,
    out_type=(_OUT, _OUT),
    scratch_types=_SCRATCH + [pltpu.SemaphoreType.DMA] * (2 * NBUF),
)


@jax.jit
def kernel(state_one_hot, edges, edge_types, W_state, b_state, edge_emb,
           W0, b0, W1, b1):
    wst_t = W_state.T
    w0a_t = W0[:, :D].T
    w0b_t = W0[:, D:].T
    w1a_t = W1[:, :D].T
    w1b_t = W1[:, D:].T
    bs2 = b_state.reshape(1, D)
    b0_2 = b0.reshape(1, D)
    b1_2 = b1.reshape(1, D)

    src = edges[0, :, 0].astype(jnp.int32)
    dst = edges[0, :, 1].astype(jnp.int32)
    et = edge_types[0].astype(jnp.int32)

    pad = PE - E
    src2 = jnp.concatenate(
        [src, jnp.full((pad,), N, jnp.int32)]).reshape(NW, G, CH)
    et2 = jnp.concatenate(
        [et, jnp.full((pad,), R, jnp.int32)]).reshape(NW, G, CH)
    dst2 = jnp.concatenate(
        [dst, jnp.zeros((pad,), jnp.int32)]).reshape(NW, G, CH)

    ident = jnp.concatenate(
        [jnp.eye(R, D, dtype=jnp.float32), jnp.zeros((1, D), jnp.float32)])

    etp = _etp_tables(edge_emb, w0b_t, b0_2, w1b_t, b1_2)
    zpad = jnp.zeros((D - R, D), jnp.float32)
    etp0 = jnp.concatenate([etp[:R], zpad], axis=0)
    etp1 = jnp.concatenate([etp[R:], zpad], axis=0)
    zrow = jnp.zeros((1, D), jnp.float32)

    p0 = _state_proj(state_one_hot, wst_t, bs2, w0a_t)
    q0 = jnp.concatenate([p0, zrow], axis=0)
    deg, acc1 = _FUSED_SCATTER(ident, q0, et2, src2, dst2)

    p1 = _mid_proj(acc1, deg, etp0, w1a_t)
    q1 = jnp.concatenate([p1, zrow], axis=0)
    acc2 = _EDGE_SCATTER(q1, src2, dst2)

    node_embeddings = _fin_sum(acc2, deg, etp1)
    return (node_embeddings, node_embeddings[0])
